# baseline (speedup 1.0000x reference)
import functools

import jax
import jax.numpy as jnp
from jax import lax
from jax.experimental import pallas as pl
from jax.experimental.pallas import tpu as pltpu
from jax.experimental.pallas import tpu_sc as plsc

_EPS = 1e-10

_NC, _NS = 2, 16
_NW = _NC * _NS
_SLOTS = 2


def _mlp_t(xt, W1, b1, W2, b2):
    h = jax.nn.relu(
        jax.lax.dot_general(W1, xt, (((0,), (0,)), ((), ())),
                            preferred_element_type=jnp.float32) + b1)
    return jax.nn.relu(
        jax.lax.dot_general(W2, h, (((0,), (0,)), ((), ())),
                            preferred_element_type=jnp.float32) + b2)


def _micro_body(xt_ref, w1n, b1n, w2n, b2n, w1s, b1s, w2s, b2s,
                x1t_ref, xsc_ref):
    xt = xt_ref[0]
    x1t_ref[0] = _mlp_t(xt, w1n[...], b1n[...], w2n[...], b2n[...])
    xs_t = _mlp_t(xt, w1s[...], b1s[...], w2s[...], b2s[...])
    xsc_ref[0] = jnp.transpose(xs_t, (1, 0))


def _make_sc_kernel(b, n, f):
    rows = b * n
    rpw = rows // _NW
    ch = n // 16
    f32 = jnp.float32

    @functools.partial(
        pl.kernel,
        out_type=[jax.ShapeDtypeStruct((rows, f, n), f32),
                  jax.ShapeDtypeStruct((rows, (f + 1) * 16), f32)],
        mesh=plsc.VectorSubcoreMesh(core_axis_name="c", subcore_axis_name="s"),
        scratch_types=[
            pltpu.VMEM((_SLOTS, f, n), f32),
            pltpu.VMEM((_SLOTS, n), f32),
            pltpu.VMEM((f, n), f32),
            pltpu.VMEM((rpw, (f + 1) * 16), f32),
            pltpu.SemaphoreType.DMA((_SLOTS,)),
            pltpu.SemaphoreType.DMA((_SLOTS,)),
            pltpu.SemaphoreType.DMA((_SLOTS,)),
        ],
    )
    def sc_kernel(a_hbm, w_hbm, x1_hbm, wout_hbm, part_hbm,
                  wbuf, abuf, x1buf, partbuf,
                  wsem, asem, osem):
        wid = lax.axis_index("s") * _NC + lax.axis_index("c")
        row0 = wid * rpw
        bi = row0 // n
        pltpu.sync_copy(x1_hbm.at[bi], x1buf)

        def win(r, slot):
            return pltpu.make_async_copy(w_hbm.at[r], wbuf.at[slot],
                                         wsem.at[slot])

        def ain(r, slot):
            return pltpu.make_async_copy(a_hbm.at[r], abuf.at[slot],
                                         asem.at[slot])

        def wout(r, slot):
            return pltpu.make_async_copy(wbuf.at[slot], wout_hbm.at[r],
                                         osem.at[slot])

        for k in range(_SLOTS):
            win(row0 + k, k).start()
            ain(row0 + k, k).start()

        def process(r, rl, slot):
            win(r, slot).wait()
            ain(r, slot).wait()

            def chunk(cix, accs):
                off = cix * 16
                av = abuf[slot, pl.ds(off, 16)]
                new = [accs[0] + av]
                for ff in range(f):
                    wv = wbuf[slot, ff, pl.ds(off, 16)]
                    xv = x1buf[ff, pl.ds(off, 16)]
                    new.append(accs[ff + 1] + av * wv * xv)
                return tuple(new)

            zero = jnp.zeros((16,), f32)
            accs = lax.fori_loop(0, ch, chunk, (zero,) * (f + 1))
            for ff in range(f):
                partbuf[rl, pl.ds(ff * 16, 16)] = accs[ff + 1]
            partbuf[rl, pl.ds(f * 16, 16)] = accs[0]
            wout(r, slot).start()

        n_iter = rpw // _SLOTS

        def outer(it, carry):
            base = row0 + it * _SLOTS
            for k in range(_SLOTS):
                process(base + k, it * _SLOTS + k, k)

            @pl.when(it + 1 < n_iter)
            def _():
                for k in range(_SLOTS):
                    wout(base + k, k).wait()
                    win(base + _SLOTS + k, k).start()
                    ain(base + _SLOTS + k, k).start()

            return carry

        lax.fori_loop(0, n_iter, outer, 0)
        for k in range(_SLOTS):
            wout(row0 + rpw - _SLOTS + k, k).wait()
        pltpu.sync_copy(partbuf, part_hbm.at[pl.ds(row0, rpw)])

    return sc_kernel


def _post_body(p_ref, xs_ref, x2_ref, *, f):
    p = p_ref[...]
    k = p.shape[1]
    sel = (jax.lax.broadcasted_iota(jnp.int32, (k, f + 1), 0) // 16
           == jax.lax.broadcasted_iota(jnp.int32, (k, f + 1), 1)).astype(
               jnp.float32)
    m = jax.lax.dot_general(p, sel, (((1,), (0,)), ((), ())),
                            preferred_element_type=jnp.float32)
    x2_ref[...] = m[:, :f] / (m[:, f:f + 1] + _EPS) + xs_ref[...]


def kernel(A, W, x, W1_n, b1_n, W2_n, b2_n, W1_s, b1_s, W2_s, b2_s):
    b, n, _, f = W.shape

    Wt = jnp.transpose(W, (0, 1, 3, 2))
    xt = jnp.transpose(x, (0, 2, 1))

    b1n = b1_n.reshape(f, 1)
    b2n = b2_n.reshape(f, 1)
    b1s = b1_s.reshape(f, 1)
    b2s = b2_s.reshape(f, 1)

    small = lambda bi: (0, 0)
    x1t_arr, xsc = pl.pallas_call(
        _micro_body,
        grid=(b,),
        in_specs=[
            pl.BlockSpec((1, f, n), lambda bi: (bi, 0, 0)),
            pl.BlockSpec((f, f), small),
            pl.BlockSpec((f, 1), small),
            pl.BlockSpec((f, f), small),
            pl.BlockSpec((f, 1), small),
            pl.BlockSpec((f, f), small),
            pl.BlockSpec((f, 1), small),
            pl.BlockSpec((f, f), small),
            pl.BlockSpec((f, 1), small),
        ],
        out_specs=[
            pl.BlockSpec((1, f, n), lambda bi: (bi, 0, 0)),
            pl.BlockSpec((1, n, f), lambda bi: (bi, 0, 0)),
        ],
        out_shape=[
            jax.ShapeDtypeStruct((b, f, n), jnp.float32),
            jax.ShapeDtypeStruct((b, n, f), jnp.float32),
        ],
    )(xt, W1_n, b1n, W2_n, b2n, W1_s, b1s, W2_s, b2s)

    a_flat = A.reshape(b * n, n)
    w_flat = Wt.reshape(b * n, f, n)
    xs_flat = xsc.reshape(b * n, f)

    wout_flat, part = _make_sc_kernel(b, n, f)(a_flat, w_flat, x1t_arr)

    rows = b * n
    bi2 = 256
    part2 = part
    x2_flat = pl.pallas_call(
        functools.partial(_post_body, f=f),
        grid=(rows // bi2,),
        in_specs=[
            pl.BlockSpec((bi2, (f + 1) * 16), lambda i: (i, 0)),
            pl.BlockSpec((bi2, f), lambda i: (i, 0)),
        ],
        out_specs=pl.BlockSpec((bi2, f), lambda i: (i, 0)),
        out_shape=jax.ShapeDtypeStruct((rows, f), jnp.float32),
    )(part2, xs_flat)

    w_out = jnp.transpose(wout_flat.reshape(b, n, f, n), (0, 1, 3, 2))
    x2 = x2_flat.reshape(b, n, f)
    return (w_out, x2)


# speedup vs baseline: 1.8987x; 1.8987x over previous
import jax
import jax.numpy as jnp
from jax.experimental import pallas as pl
from jax.experimental.pallas import tpu as pltpu

_EPS = 1e-10

_BI = 64
_NBUF = 8
_DEPTH = 4


def _mlp_t(xt, W1, b1, W2, b2):
    h = jax.nn.relu(
        jax.lax.dot_general(W1, xt, (((0,), (0,)), ((), ())),
                            preferred_element_type=jnp.float32) + b1)
    return jax.nn.relu(
        jax.lax.dot_general(W2, h, (((0,), (0,)), ((), ())),
                            preferred_element_type=jnp.float32) + b2)


def _make_body(b, n, f):
    n_blk = n // _BI
    n_step = b * n_blk

    def body(a_hbm, w_hbm, xt_vmem,
             w1n, b1n, w2n, b2n, w1s, b1s, w2s, b2s,
             w_out_hbm, x2_vmem,
             wbuf, abuf, x1_vmem, xs_vmem, in_sem, a_in_sem, out_sem):
        for bi in range(b):
            xt = xt_vmem[bi]
            x1_vmem[bi] = _mlp_t(xt, w1n[...], b1n[...], w2n[...], b2n[...])
            xs_vmem[bi] = _mlp_t(xt, w1s[...], b1s[...], w2s[...], b2s[...])

        def start_in(t):
            slot = t % _NBUF
            bi, blk = divmod(t, n_blk)
            i0 = blk * _BI
            pltpu.make_async_copy(w_hbm.at[bi, pl.ds(i0, _BI)],
                                  wbuf.at[slot], in_sem.at[slot]).start()
            pltpu.make_async_copy(a_hbm.at[bi, pl.ds(i0, _BI)],
                                  abuf.at[slot], a_in_sem.at[slot]).start()

        def out_copy(t):
            slot = t % _NBUF
            bi, blk = divmod(t, n_blk)
            i0 = blk * _BI
            return pltpu.make_async_copy(wbuf.at[slot],
                                         w_out_hbm.at[bi, pl.ds(i0, _BI)],
                                         out_sem.at[slot])

        for t in range(min(_DEPTH, n_step)):
            start_in(t)

        for t in range(n_step):
            slot = t % _NBUF
            bi, blk = divmod(t, n_blk)
            i0 = blk * _BI

            pltpu.make_async_copy(w_hbm.at[bi, pl.ds(i0, _BI)],
                                  wbuf.at[slot], in_sem.at[slot]).wait()
            pltpu.make_async_copy(a_hbm.at[bi, pl.ds(i0, _BI)],
                                  abuf.at[slot], a_in_sem.at[slot]).wait()

            a = abuf[slot]
            a_sum = jnp.sum(a, axis=1, keepdims=True) + _EPS
            w = wbuf[slot]
            q = w * x1_vmem[bi][None, :, :] * a[:, None, :]
            m = jnp.sum(q, axis=2) / a_sum
            x2_vmem[bi, :, i0:i0 + _BI] = xs_vmem[bi, :, i0:i0 + _BI] + m.T

            out_copy(t).start()

            nxt = t + _DEPTH
            if nxt < n_step:
                prev = nxt - _NBUF
                if prev >= 0:
                    out_copy(prev).wait()
                start_in(nxt)

        for t in range(max(0, n_step - _NBUF), n_step):
            out_copy(t).wait()

    return body


def kernel(A, W, x, W1_n, b1_n, W2_n, b2_n, W1_s, b1_s, W2_s, b2_s):
    b, n, _, f = W.shape

    Wt = jnp.transpose(W, (0, 1, 3, 2))
    xt = jnp.transpose(x, (0, 2, 1))

    b1n = b1_n.reshape(f, 1)
    b2n = b2_n.reshape(f, 1)
    b1s = b1_s.reshape(f, 1)
    b2s = b2_s.reshape(f, 1)

    vsmall = pl.BlockSpec(memory_space=pltpu.VMEM)
    any_ = pl.BlockSpec(memory_space=pl.ANY)

    w_out, x2t = pl.pallas_call(
        _make_body(b, n, f),
        in_specs=[any_, any_, vsmall,
                  vsmall, vsmall, vsmall, vsmall,
                  vsmall, vsmall, vsmall, vsmall],
        out_specs=[any_, vsmall],
        out_shape=[
            jax.ShapeDtypeStruct((b, n, f, n), W.dtype),
            jax.ShapeDtypeStruct((b, f, n), x.dtype),
        ],
        scratch_shapes=[
            pltpu.VMEM((_NBUF, _BI, f, n), jnp.float32),
            pltpu.VMEM((_NBUF, _BI, n), jnp.float32),
            pltpu.VMEM((b, f, n), jnp.float32),
            pltpu.VMEM((b, f, n), jnp.float32),
            pltpu.SemaphoreType.DMA((_NBUF,)),
            pltpu.SemaphoreType.DMA((_NBUF,)),
            pltpu.SemaphoreType.DMA((_NBUF,)),
        ],
    )(A, Wt, xt, W1_n, b1n, W2_n, b2n, W1_s, b1s, W2_s, b2s)
    return (jnp.transpose(w_out, (0, 1, 3, 2)), jnp.transpose(x2t, (0, 2, 1)))
